# Initial kernel scaffold; baseline (speedup 1.0000x reference)
import jax
import jax.numpy as jnp
from jax.experimental import pallas as pl


def kernel(x, edge_index, edge_attr, batch, W1, b1, g1, be1, m1, v1, W2, b2, g2, be2, m2, v2, Wc1, bc1, Wc2, bc2):
    raise NotImplementedError("write your pallas kernel here")


# speedup vs baseline: 8.8131x; 8.8131x over previous
import functools

import jax
import jax.numpy as jnp
from jax import lax
from jax.experimental import pallas as pl
from jax.experimental.pallas import tpu as pltpu
from jax.experimental.pallas import tpu_sc as plsc

_N = 10000
_E = 320000
_D = 128
_H = 64
_G = 64
_C = 2

_NC = 2
_NS = 16
_NT = _NC * _NS
_CH = 128
_NCH = 80
_EPAD = _NT * _NCH * _CH
_NP = 10240
_RPS = 640

_mesh = plsc.VectorSubcoreMesh(core_axis_name="c", subcore_axis_name="s")


@functools.partial(
    pl.kernel,
    out_type=jax.ShapeDtypeStruct((_NC, _NP, 16), jnp.float32),
    mesh=_mesh,
    compiler_params=pltpu.CompilerParams(use_tc_tiling_on_sc=False),
    scratch_types=[
        pltpu.VMEM((_NCH, _CH), jnp.int32),
        pltpu.VMEM((_NCH, _CH), jnp.float32),
        pltpu.VMEM((_CH, 16), jnp.float32),
        pltpu.VMEM_SHARED((_NP, 16), jnp.float32),
    ],
)
def _sc_deg(dst_hbm, ew_hbm, out_hbm, dst_v, ew_v, evbuf, acc_sh):
    c = lax.axis_index("c")
    s = lax.axis_index("s")
    tid = c * _NS + s

    zero16 = jnp.zeros((16,), jnp.float32)

    def _zero_evbuf(i, _):
        evbuf[i, pl.ds(0, 16)] = zero16
        return 0

    lax.fori_loop(0, _CH, _zero_evbuf, 0)

    base = s * _RPS
    for k in range(5):
        pltpu.sync_copy(evbuf, acc_sh.at[pl.ds(base + k * _CH, _CH)])
    plsc.subcore_barrier()

    pltpu.sync_copy(dst_hbm.at[tid], dst_v)
    pltpu.sync_copy(ew_hbm.at[tid], ew_v)

    def _chunk(ci, _):
        def _fill(g, _):
            wv = ew_v[ci, pl.ds(g * 16, 16)]
            for r in range(16):
                evbuf[g * 16 + r, pl.ds(0, 16)] = jnp.broadcast_to(wv[r], (16,))
            return 0

        lax.fori_loop(0, _CH // 16, _fill, 0)
        pltpu.sync_copy(evbuf, acc_sh.at[dst_v.at[ci]], add=True)
        return 0

    lax.fori_loop(0, _NCH, _chunk, 0)
    plsc.subcore_barrier()

    for k in range(5):
        pltpu.sync_copy(acc_sh.at[pl.ds(base + k * _CH, _CH)], evbuf)
        pltpu.sync_copy(evbuf, out_hbm.at[c, pl.ds(base + k * _CH, _CH)])


@functools.partial(
    pl.kernel,
    out_type=jax.ShapeDtypeStruct((_NC, _NP, _H), jnp.float32),
    mesh=_mesh,
    compiler_params=pltpu.CompilerParams(use_tc_tiling_on_sc=False),
    scratch_types=[
        pltpu.VMEM((_NCH, _CH), jnp.int32),
        pltpu.VMEM((_NCH, _CH), jnp.int32),
        pltpu.VMEM((_NCH, _CH), jnp.float32),
        pltpu.VMEM((_CH, _H), jnp.float32),
        pltpu.VMEM_SHARED((_NP, _H), jnp.float32),
        pltpu.SemaphoreType.DMA,
    ],
)
def _sc_agg(g_hbm, src_hbm, dst_hbm, ew_hbm, out_hbm,
            src_v, dst_v, ew_v, rows, acc_sh, sem):
    c = lax.axis_index("c")
    s = lax.axis_index("s")
    tid = c * _NS + s

    zero16 = jnp.zeros((16,), jnp.float32)

    def _zero_rows(i, _):
        for j in range(4):
            rows[i, pl.ds(j * 16, 16)] = zero16
        return 0

    lax.fori_loop(0, _CH, _zero_rows, 0)

    base = s * _RPS
    for k in range(5):
        pltpu.sync_copy(rows, acc_sh.at[pl.ds(base + k * _CH, _CH)])
    plsc.subcore_barrier()

    pltpu.sync_copy(src_hbm.at[tid], src_v)
    pltpu.sync_copy(dst_hbm.at[tid], dst_v)
    pltpu.sync_copy(ew_hbm.at[tid], ew_v)

    def _chunk(ci, _):
        pltpu.async_copy(g_hbm.at[src_v.at[ci]], rows, sem).wait()

        def _scale(g, _):
            wv = ew_v[ci, pl.ds(g * 16, 16)]
            for r in range(16):
                i = g * 16 + r
                w = wv[r]
                for j in range(4):
                    sl = pl.ds(j * 16, 16)
                    rows[i, sl] = rows[i, sl] * w
            return 0

        lax.fori_loop(0, _CH // 16, _scale, 0)
        pltpu.sync_copy(rows, acc_sh.at[dst_v.at[ci]], add=True)
        return 0

    lax.fori_loop(0, _NCH, _chunk, 0)
    plsc.subcore_barrier()

    for k in range(5):
        pltpu.sync_copy(acc_sh.at[pl.ds(base + k * _CH, _CH)], rows)
        pltpu.sync_copy(rows, out_hbm.at[c, pl.ds(base + k * _CH, _CH)])


def _tc1_body(degp_ref, x_ref, w1_ref, dinv_ref, h1_ref, g1_ref):
    degp = degp_ref[:]
    deg = degp[0, :_N, 0] + degp[1, :_N, 0] + 1.0
    dinv = jnp.where(deg > 0, 1.0 / jnp.sqrt(jnp.maximum(deg, 1e-12)), 0.0)
    dinv_ref[:] = dinv
    h1 = jnp.dot(x_ref[:], w1_ref[:], preferred_element_type=jnp.float32)
    h1_ref[:] = h1
    g1_ref[:] = h1 * dinv[:, None]


_tc1 = pl.pallas_call(
    _tc1_body,
    out_shape=[
        jax.ShapeDtypeStruct((_N,), jnp.float32),
        jax.ShapeDtypeStruct((_N, _H), jnp.float32),
        jax.ShapeDtypeStruct((_N, _H), jnp.float32),
    ],
)


def _tc2_body(acc_ref, h1_ref, dinv_ref, b1_ref, g1_ref, be1_ref, m1_ref,
              v1_ref, w2_ref, h2_ref, g2_ref):
    dinv = dinv_ref[:]
    a = acc_ref[:]
    acc = a[0, :_N] + a[1, :_N]
    a1 = acc * dinv[:, None] + h1_ref[:] * (dinv * dinv)[:, None] + b1_ref[:]
    h1p = (a1 - m1_ref[:]) / jnp.sqrt(v1_ref[:] + 1e-5) * g1_ref[:] + be1_ref[:]
    h1p = jnp.maximum(h1p, 0.0)
    h2 = jnp.dot(h1p, w2_ref[:], preferred_element_type=jnp.float32)
    h2_ref[:] = h2
    g2_ref[:] = h2 * dinv[:, None]


_tc2 = pl.pallas_call(
    _tc2_body,
    out_shape=[
        jax.ShapeDtypeStruct((_N, _H), jnp.float32),
        jax.ShapeDtypeStruct((_N, _H), jnp.float32),
    ],
)


def _tc3_body(acc_ref, h2_ref, dinv_ref, b2_ref, g2_ref, be2_ref, m2_ref,
              v2_ref, batch_ref, wc1_ref, bc1_ref, wc2_ref, bc2_ref, out_ref):
    dinv = dinv_ref[:]
    a = acc_ref[:]
    acc = a[0, :_N] + a[1, :_N]
    a2 = acc * dinv[:, None] + h2_ref[:] * (dinv * dinv)[:, None] + b2_ref[:]
    h2p = (a2 - m2_ref[:]) / jnp.sqrt(v2_ref[:] + 1e-5) * g2_ref[:] + be2_ref[:]
    h2p = jnp.maximum(h2p, 0.0)
    onehot = (batch_ref[:][:, None]
              == lax.broadcasted_iota(jnp.int32, (_N, _G), 1)).astype(jnp.float32)
    sums = lax.dot_general(onehot, h2p, (((0,), (0,)), ((), ())),
                           preferred_element_type=jnp.float32)
    cnt = jnp.sum(onehot, axis=0)
    pooled = sums / jnp.maximum(cnt, 1.0)[:, None]
    z = jnp.maximum(jnp.dot(pooled, wc1_ref[:],
                            preferred_element_type=jnp.float32) + bc1_ref[:], 0.0)
    out_ref[:] = jnp.dot(z, wc2_ref[:],
                         preferred_element_type=jnp.float32) + bc2_ref[:]


_tc3 = pl.pallas_call(
    _tc3_body,
    out_shape=jax.ShapeDtypeStruct((_G, _C), jnp.float32),
)


def kernel(x, edge_index, edge_attr, batch, W1, b1, g1, be1, m1, v1,
           W2, b2, g2, be2, m2, v2, Wc1, bc1, Wc2, bc2):
    src = edge_index[0]
    dst = edge_index[1]
    ew = edge_attr[:, 0]
    pad = _EPAD - _E
    srcp = jnp.pad(src, (0, pad)).reshape(_NT, _NCH, _CH)
    dstp = jnp.pad(dst, (0, pad)).reshape(_NT, _NCH, _CH)
    ewp = jnp.pad(ew, (0, pad)).reshape(_NT, _NCH, _CH)

    degp = _sc_deg(dstp, ewp)
    dinv, h1, g1l = _tc1(degp, x, W1)
    acc1 = _sc_agg(g1l, srcp, dstp, ewp)
    h2, g2l = _tc2(acc1, h1, dinv, b1, g1, be1, m1, v1, W2)
    acc2 = _sc_agg(g2l, srcp, dstp, ewp)
    return _tc3(acc2, h2, dinv, b2, g2, be2, m2, v2, batch, Wc1, bc1, Wc2, bc2)


# speedup vs baseline: 12.9990x; 1.4750x over previous
import functools

import jax
import jax.numpy as jnp
from jax import lax
from jax.experimental import pallas as pl
from jax.experimental.pallas import tpu as pltpu
from jax.experimental.pallas import tpu_sc as plsc

_N = 10000
_E = 320000
_D = 128
_H = 64
_G = 64
_C = 2

_NC = 2
_NS = 16
_NT = _NC * _NS
_CH = 128
_NCH = 80
_EPAD = _NT * _NCH * _CH
_NP = 10240
_RPS = 640

_mesh = plsc.VectorSubcoreMesh(core_axis_name="c", subcore_axis_name="s")


@functools.partial(
    pl.kernel,
    out_type=jax.ShapeDtypeStruct((_NC, _NP, 16), jnp.float32),
    mesh=_mesh,
    compiler_params=pltpu.CompilerParams(use_tc_tiling_on_sc=False),
    scratch_types=[
        pltpu.VMEM((_NCH, _CH), jnp.int32),
        pltpu.VMEM((_NCH, _CH), jnp.float32),
        pltpu.VMEM((_CH, 16), jnp.float32),
        pltpu.VMEM_SHARED((_NP, 16), jnp.float32),
    ],
)
def _sc_deg(dst_hbm, ew_hbm, out_hbm, dst_v, ew_v, evbuf, acc_sh):
    c = lax.axis_index("c")
    s = lax.axis_index("s")
    tid = c * _NS + s

    zero16 = jnp.zeros((16,), jnp.float32)

    def _zero_evbuf(i, _):
        evbuf[i, pl.ds(0, 16)] = zero16
        return 0

    lax.fori_loop(0, _CH, _zero_evbuf, 0)

    base = s * _RPS
    for k in range(5):
        pltpu.sync_copy(evbuf, acc_sh.at[pl.ds(base + k * _CH, _CH)])
    plsc.subcore_barrier()

    pltpu.sync_copy(dst_hbm.at[tid], dst_v)
    pltpu.sync_copy(ew_hbm.at[tid], ew_v)

    def _chunk(ci, _):
        def _fill(g, _):
            wv = ew_v[ci, pl.ds(g * 16, 16)]
            for r in range(16):
                evbuf[g * 16 + r, pl.ds(0, 16)] = jnp.broadcast_to(wv[r], (16,))
            return 0

        lax.fori_loop(0, _CH // 16, _fill, 0)
        pltpu.sync_copy(evbuf, acc_sh.at[dst_v.at[ci]], add=True)
        return 0

    lax.fori_loop(0, _NCH, _chunk, 0)
    plsc.subcore_barrier()

    for k in range(5):
        pltpu.sync_copy(acc_sh.at[pl.ds(base + k * _CH, _CH)], evbuf)
        pltpu.sync_copy(evbuf, out_hbm.at[c, pl.ds(base + k * _CH, _CH)])


@functools.partial(
    pl.kernel,
    out_type=jax.ShapeDtypeStruct((_NC, _NP, _H), jnp.float32),
    mesh=_mesh,
    compiler_params=pltpu.CompilerParams(use_tc_tiling_on_sc=False),
    scratch_types=[
        pltpu.VMEM((_NCH, _CH), jnp.int32),
        pltpu.VMEM((_NCH, _CH), jnp.int32),
        pltpu.VMEM((_NCH, _CH), jnp.float32),
        pltpu.VMEM((_CH, _H), jnp.float32),
        pltpu.VMEM((_CH, _H), jnp.float32),
        pltpu.VMEM((_CH, _H), jnp.float32),
        pltpu.VMEM((_CH, _H), jnp.float32),
        pltpu.VMEM_SHARED((_NP, _H), jnp.float32),
        pltpu.SemaphoreType.DMA,
        pltpu.SemaphoreType.DMA,
        pltpu.SemaphoreType.DMA,
        pltpu.SemaphoreType.DMA,
        pltpu.SemaphoreType.DMA,
        pltpu.SemaphoreType.DMA,
        pltpu.SemaphoreType.DMA,
        pltpu.SemaphoreType.DMA,
    ],
)
def _sc_agg(g_hbm, src_hbm, dst_hbm, ew_hbm, out_hbm,
            src_v, dst_v, ew_v, rows0, rows1, rows2, rows3, acc_sh,
            gsem0, gsem1, gsem2, gsem3, ssem0, ssem1, ssem2, ssem3):
    c = lax.axis_index("c")
    s = lax.axis_index("s")
    tid = c * _NS + s
    bufs = (rows0, rows1, rows2, rows3)
    gsems = (gsem0, gsem1, gsem2, gsem3)
    ssems = (ssem0, ssem1, ssem2, ssem3)

    zero16 = jnp.zeros((16,), jnp.float32)

    def _zero_rows(i, _):
        for j in range(4):
            rows0[i, pl.ds(j * 16, 16)] = zero16
        return 0

    lax.fori_loop(0, _CH, _zero_rows, 0)

    base = s * _RPS
    for k in range(5):
        pltpu.sync_copy(rows0, acc_sh.at[pl.ds(base + k * _CH, _CH)])
    plsc.subcore_barrier()

    pltpu.sync_copy(src_hbm.at[tid], src_v)
    pltpu.sync_copy(dst_hbm.at[tid], dst_v)
    pltpu.sync_copy(ew_hbm.at[tid], ew_v)

    for k in range(2):
        pltpu.async_copy(g_hbm.at[src_v.at[k]], bufs[k], gsems[k])

    def _outer(o, _):
        base_ci = o * 4
        for k in range(4):
            ci = base_ci + k
            nb = (k + 2) % 4
            nci = ci + 2

            @pl.when(nci < _NCH)
            def _start_next():
                pltpu.async_copy(g_hbm.at[src_v.at[nci]], bufs[nb], gsems[nb])

            pltpu.make_async_copy(
                g_hbm.at[src_v.at[ci]], bufs[k], gsems[k]
            ).wait()

            rows = bufs[k]

            def _scale(g, _):
                wv = ew_v[ci, pl.ds(g * 16, 16)]
                for r in range(16):
                    i = g * 16 + r
                    w = wv[r]
                    for j in range(4):
                        sl = pl.ds(j * 16, 16)
                        rows[i, sl] = rows[i, sl] * w
                return 0

            lax.fori_loop(0, _CH // 16, _scale, 0)
            pltpu.sync_copy(rows, acc_sh.at[dst_v.at[ci]], add=True)
        return 0

    lax.fori_loop(0, _NCH // 4, _outer, 0)
    plsc.subcore_barrier()

    for k in range(5):
        pltpu.sync_copy(acc_sh.at[pl.ds(base + k * _CH, _CH)], rows0)
        pltpu.sync_copy(rows0, out_hbm.at[c, pl.ds(base + k * _CH, _CH)])


def _tc1_body(degp_ref, x_ref, w1_ref, dinv_ref, h1_ref, g1_ref):
    degp = degp_ref[:]
    deg = degp[0, :_N, 0] + degp[1, :_N, 0] + 1.0
    dinv = jnp.where(deg > 0, 1.0 / jnp.sqrt(jnp.maximum(deg, 1e-12)), 0.0)
    dinv_ref[:] = dinv
    h1 = jnp.dot(x_ref[:], w1_ref[:], preferred_element_type=jnp.float32)
    h1_ref[:] = h1
    g1_ref[:] = h1 * dinv[:, None]


_tc1 = pl.pallas_call(
    _tc1_body,
    out_shape=[
        jax.ShapeDtypeStruct((_N,), jnp.float32),
        jax.ShapeDtypeStruct((_N, _H), jnp.float32),
        jax.ShapeDtypeStruct((_N, _H), jnp.float32),
    ],
)


def _tc2_body(acc_ref, h1_ref, dinv_ref, b1_ref, g1_ref, be1_ref, m1_ref,
              v1_ref, w2_ref, h2_ref, g2_ref):
    dinv = dinv_ref[:]
    a = acc_ref[:]
    acc = a[0, :_N] + a[1, :_N]
    a1 = acc * dinv[:, None] + h1_ref[:] * (dinv * dinv)[:, None] + b1_ref[:]
    h1p = (a1 - m1_ref[:]) / jnp.sqrt(v1_ref[:] + 1e-5) * g1_ref[:] + be1_ref[:]
    h1p = jnp.maximum(h1p, 0.0)
    h2 = jnp.dot(h1p, w2_ref[:], preferred_element_type=jnp.float32)
    h2_ref[:] = h2
    g2_ref[:] = h2 * dinv[:, None]


_tc2 = pl.pallas_call(
    _tc2_body,
    out_shape=[
        jax.ShapeDtypeStruct((_N, _H), jnp.float32),
        jax.ShapeDtypeStruct((_N, _H), jnp.float32),
    ],
)


def _tc3_body(acc_ref, h2_ref, dinv_ref, b2_ref, g2_ref, be2_ref, m2_ref,
              v2_ref, batch_ref, wc1_ref, bc1_ref, wc2_ref, bc2_ref, out_ref):
    dinv = dinv_ref[:]
    a = acc_ref[:]
    acc = a[0, :_N] + a[1, :_N]
    a2 = acc * dinv[:, None] + h2_ref[:] * (dinv * dinv)[:, None] + b2_ref[:]
    h2p = (a2 - m2_ref[:]) / jnp.sqrt(v2_ref[:] + 1e-5) * g2_ref[:] + be2_ref[:]
    h2p = jnp.maximum(h2p, 0.0)
    onehot = (batch_ref[:][:, None]
              == lax.broadcasted_iota(jnp.int32, (_N, _G), 1)).astype(jnp.float32)
    sums = lax.dot_general(onehot, h2p, (((0,), (0,)), ((), ())),
                           preferred_element_type=jnp.float32,
                           precision=lax.Precision.HIGHEST)
    cnt = jnp.sum(onehot, axis=0)
    pooled = sums / jnp.maximum(cnt, 1.0)[:, None]
    z = jnp.maximum(jnp.dot(pooled, wc1_ref[:],
                            preferred_element_type=jnp.float32) + bc1_ref[:], 0.0)
    out_ref[:] = jnp.dot(z, wc2_ref[:],
                         preferred_element_type=jnp.float32) + bc2_ref[:]


_tc3 = pl.pallas_call(
    _tc3_body,
    out_shape=jax.ShapeDtypeStruct((_G, _C), jnp.float32),
)


def kernel(x, edge_index, edge_attr, batch, W1, b1, g1, be1, m1, v1,
           W2, b2, g2, be2, m2, v2, Wc1, bc1, Wc2, bc2):
    src = edge_index[0]
    dst = edge_index[1]
    ew = edge_attr[:, 0]
    pad = _EPAD - _E
    srcp = jnp.pad(src, (0, pad)).reshape(_NT, _NCH, _CH)
    dstp = jnp.pad(dst, (0, pad)).reshape(_NT, _NCH, _CH)
    ewp = jnp.pad(ew, (0, pad)).reshape(_NT, _NCH, _CH)

    degp = _sc_deg(dstp, ewp)
    dinv, h1, g1l = _tc1(degp, x, W1)
    acc1 = _sc_agg(g1l, srcp, dstp, ewp)
    h2, g2l = _tc2(acc1, h1, dinv, b1, g1, be1, m1, v1, W2)
    acc2 = _sc_agg(g2l, srcp, dstp, ewp)
    return _tc3(acc2, h2, dinv, b2, g2, be2, m2, v2, batch, Wc1, bc1, Wc2, bc2)


def kernel(x, edge_index, edge_attr, batch, W1, b1, g1, be1, m1, v1,
           W2, b2, g2, be2, m2, v2, Wc1, bc1, Wc2, bc2):
    src = edge_index[0]
    dst = edge_index[1]
    ew = edge_attr[:, 0]
    pad = _EPAD - _E
    srcp = jnp.pad(src, (0, pad)).reshape(_NT, _NCH, _CH)
    dstp = jnp.pad(dst, (0, pad)).reshape(_NT, _NCH, _CH)
    ewp = jnp.pad(ew, (0, pad)).reshape(_NT, _NCH, _CH)

    degp = _sc_deg(dstp, ewp)
    dinv, h1, g1l = _tc1(degp, x, W1)
    acc1 = _sc_agg(g1l, srcp, dstp, ewp)
    h2, g2l = _tc2(acc1, h1, dinv, b1, g1, be1, m1, v1, W2)
    acc2 = _sc_agg(g2l, srcp, dstp, ewp)
    return _tc3(acc2, h2, dinv, b2, g2, be2, m2, v2, batch, Wc1, bc1, Wc2, bc2)
